# Initial kernel scaffold; baseline (speedup 1.0000x reference)
import jax
import jax.numpy as jnp
from jax.experimental import pallas as pl


def kernel(cos, sin, hidden_states, active_mask, Wqkv, Wo):
    raise NotImplementedError("write your pallas kernel here")


# speedup vs baseline: 2.1816x; 2.1816x over previous
import functools

import jax
import jax.numpy as jnp
from jax.experimental import pallas as pl
from jax.experimental.pallas import tpu as pltpu

B, L, D_MODEL = 1, 2048, 768
H, KV, HD = 12, 12, 64
OUT = H * HD
BLK = 256
NBLK = L // BLK
SCALE = 1.0 / (HD ** 0.5)


def _rotary_cols(x, cos_c, sin_c):
    rl = jnp.concatenate([x[:, 32:], x[:, :32]], axis=1)
    rr = jnp.concatenate([x[:, -32:], x[:, :-32]], axis=1)
    lane = jax.lax.broadcasted_iota(jnp.int32, x.shape, 1)
    first_half = (lane % HD) < (HD // 2)
    roth = jnp.where(first_half, -rl, rr)
    return x * cos_c + roth * sin_c


def _rotary_rows(x, cos_c, sin_c):
    rl = jnp.concatenate([x[32:, :], x[:32, :]], axis=0)
    rr = jnp.concatenate([x[-32:, :], x[:-32, :]], axis=0)
    sub = jax.lax.broadcasted_iota(jnp.int32, x.shape, 0)
    first_half = (sub % HD) < (HD // 2)
    roth = jnp.where(first_half, -rl, rr)
    return x * cos_c + roth * sin_c


def _qkv_body(x_ref, xt_ref, wqt_ref, wk_ref, wvt_ref,
              cos_ref, sin_ref, cost_ref, sint_ref,
              q_ref, kt_ref, v_ref):
    xb = x_ref[...].astype(jnp.bfloat16)
    q = jnp.dot(xb, wqt_ref[...], preferred_element_type=jnp.float32)
    cos_c = jnp.concatenate([cos_ref[...]] * H, axis=1)
    sin_c = jnp.concatenate([sin_ref[...]] * H, axis=1)
    q_ref[...] = _rotary_cols(q, cos_c, sin_c)
    v = jnp.dot(xb, wvt_ref[...], preferred_element_type=jnp.float32)
    v_ref[...] = v.astype(jnp.bfloat16)
    xtb = xt_ref[...].astype(jnp.bfloat16)
    kt = jnp.dot(wk_ref[...], xtb, preferred_element_type=jnp.float32)
    cost_c = jnp.concatenate([cost_ref[...]] * H, axis=0)
    sint_c = jnp.concatenate([sint_ref[...]] * H, axis=0)
    kt_ref[...] = _rotary_rows(kt, cost_c, sint_c).astype(jnp.bfloat16)


def _attn_body(q_ref, kt_ref, v_ref, wot_ref, mask_ref, out_ref, acc_ref):
    qb = q_ref[...].astype(jnp.bfloat16)
    for h in range(H):
        sl = slice(HD * h, HD * (h + 1))
        s = jnp.dot(qb[:, sl], kt_ref[sl, :],
                    preferred_element_type=jnp.float32)
        p = jnp.exp(s)
        denom = jnp.sum(p, axis=1, keepdims=True)
        o = jnp.dot(p.astype(jnp.bfloat16), v_ref[:, sl],
                    preferred_element_type=jnp.float32)
        acc_ref[:, sl] = o / denom
    ob = acc_ref[...].astype(jnp.bfloat16)
    out = jnp.dot(ob, wot_ref[...], preferred_element_type=jnp.float32)
    out_ref[...] = out * mask_ref[...]


@jax.jit
def kernel(cos, sin, hidden_states, active_mask, Wqkv, Wo):
    x = hidden_states[0]
    xt = x.T
    wqt = (Wqkv[:OUT].T * SCALE).astype(jnp.bfloat16)
    wk = Wqkv[OUT:2 * OUT].astype(jnp.bfloat16)
    wvt = Wqkv[2 * OUT:].T.astype(jnp.bfloat16)
    wot = Wo.T.astype(jnp.bfloat16)
    cos2, sin2 = cos[0], sin[0]
    cost, sint = cos2.T, sin2.T
    maskf = active_mask[0].astype(jnp.float32).reshape(L, 1)

    q, kt, v = pl.pallas_call(
        _qkv_body,
        grid=(NBLK,),
        in_specs=[
            pl.BlockSpec((BLK, D_MODEL), lambda i: (i, 0)),
            pl.BlockSpec((D_MODEL, BLK), lambda i: (0, i)),
            pl.BlockSpec((D_MODEL, OUT), lambda i: (0, 0)),
            pl.BlockSpec((OUT, D_MODEL), lambda i: (0, 0)),
            pl.BlockSpec((D_MODEL, OUT), lambda i: (0, 0)),
            pl.BlockSpec((BLK, HD), lambda i: (i, 0)),
            pl.BlockSpec((BLK, HD), lambda i: (i, 0)),
            pl.BlockSpec((HD, BLK), lambda i: (0, i)),
            pl.BlockSpec((HD, BLK), lambda i: (0, i)),
        ],
        out_specs=[
            pl.BlockSpec((BLK, OUT), lambda i: (i, 0)),
            pl.BlockSpec((OUT, BLK), lambda i: (0, i)),
            pl.BlockSpec((BLK, OUT), lambda i: (i, 0)),
        ],
        out_shape=[
            jax.ShapeDtypeStruct((L, OUT), jnp.float32),
            jax.ShapeDtypeStruct((OUT, L), jnp.bfloat16),
            jax.ShapeDtypeStruct((L, OUT), jnp.bfloat16),
        ],
    )(x, xt, wqt, wk, wvt, cos2, sin2, cost, sint)

    out = pl.pallas_call(
        _attn_body,
        grid=(NBLK,),
        in_specs=[
            pl.BlockSpec((BLK, OUT), lambda i: (i, 0)),
            pl.BlockSpec((OUT, L), lambda i: (0, 0)),
            pl.BlockSpec((L, OUT), lambda i: (0, 0)),
            pl.BlockSpec((OUT, OUT), lambda i: (0, 0)),
            pl.BlockSpec((BLK, 1), lambda i: (i, 0)),
        ],
        out_specs=pl.BlockSpec((BLK, OUT), lambda i: (i, 0)),
        out_shape=jax.ShapeDtypeStruct((L, OUT), jnp.float32),
        scratch_shapes=[pltpu.VMEM((BLK, OUT), jnp.float32)],
    )(q, kt, v, wot, maskf)

    return out.reshape(B, L, OUT)


# speedup vs baseline: 2.2481x; 1.0305x over previous
import functools

import jax
import jax.numpy as jnp
from jax.experimental import pallas as pl
from jax.experimental.pallas import tpu as pltpu

B, L, D_MODEL = 1, 2048, 768
H, KV, HD = 12, 12, 64
OUT = H * HD
BLK = 256
NBLK = L // BLK
SCALE = 1.0 / (HD ** 0.5)


def _rotary_cols(x, cos_c, sin_c):
    rl = jnp.concatenate([x[:, 32:], x[:, :32]], axis=1)
    rr = jnp.concatenate([x[:, -32:], x[:, :-32]], axis=1)
    lane = jax.lax.broadcasted_iota(jnp.int32, x.shape, 1)
    first_half = (lane % HD) < (HD // 2)
    roth = jnp.where(first_half, -rl, rr)
    return x * cos_c + roth * sin_c


def _qkv_body(x_ref, wqt_ref, wkt_ref, wvt_ref, cos_ref, sin_ref,
              q_ref, k_ref, v_ref):
    xb = x_ref[...].astype(jnp.bfloat16)
    cos_c = jnp.concatenate([cos_ref[...]] * H, axis=1)
    sin_c = jnp.concatenate([sin_ref[...]] * H, axis=1)
    q = jnp.dot(xb, wqt_ref[...], preferred_element_type=jnp.float32)
    q_ref[...] = _rotary_cols(q, cos_c, sin_c)
    k = jnp.dot(xb, wkt_ref[...], preferred_element_type=jnp.float32)
    k_ref[...] = _rotary_cols(k, cos_c, sin_c).astype(jnp.bfloat16)
    v = jnp.dot(xb, wvt_ref[...], preferred_element_type=jnp.float32)
    v_ref[...] = v.astype(jnp.bfloat16)


def _attn_body(q_ref, k_ref, v_ref, wot_ref, mask_ref, out_ref, acc_ref):
    qb = q_ref[...].astype(jnp.bfloat16)
    for h in range(H):
        sl = slice(HD * h, HD * (h + 1))
        s = jax.lax.dot_general(
            qb[:, sl], k_ref[:, sl], (((1,), (1,)), ((), ())),
            preferred_element_type=jnp.float32)
        p = jnp.exp(s)
        denom = jnp.sum(p, axis=1, keepdims=True)
        o = jnp.dot(p.astype(jnp.bfloat16), v_ref[:, sl],
                    preferred_element_type=jnp.float32)
        acc_ref[:, sl] = o / denom
    ob = acc_ref[...].astype(jnp.bfloat16)
    out = jnp.dot(ob, wot_ref[...], preferred_element_type=jnp.float32)
    out_ref[...] = out * mask_ref[...]


@jax.jit
def kernel(cos, sin, hidden_states, active_mask, Wqkv, Wo):
    x = hidden_states[0]
    wqt = (Wqkv[:OUT].T * SCALE).astype(jnp.bfloat16)
    wkt = Wqkv[OUT:2 * OUT].T.astype(jnp.bfloat16)
    wvt = Wqkv[2 * OUT:].T.astype(jnp.bfloat16)
    wot = Wo.T.astype(jnp.bfloat16)
    cos2, sin2 = cos[0], sin[0]
    maskf = active_mask[0].astype(jnp.float32).reshape(L, 1)

    q, k, v = pl.pallas_call(
        _qkv_body,
        grid=(NBLK,),
        in_specs=[
            pl.BlockSpec((BLK, D_MODEL), lambda i: (i, 0)),
            pl.BlockSpec((D_MODEL, OUT), lambda i: (0, 0)),
            pl.BlockSpec((D_MODEL, OUT), lambda i: (0, 0)),
            pl.BlockSpec((D_MODEL, OUT), lambda i: (0, 0)),
            pl.BlockSpec((BLK, HD), lambda i: (i, 0)),
            pl.BlockSpec((BLK, HD), lambda i: (i, 0)),
        ],
        out_specs=[
            pl.BlockSpec((BLK, OUT), lambda i: (i, 0)),
            pl.BlockSpec((BLK, OUT), lambda i: (i, 0)),
            pl.BlockSpec((BLK, OUT), lambda i: (i, 0)),
        ],
        out_shape=[
            jax.ShapeDtypeStruct((L, OUT), jnp.float32),
            jax.ShapeDtypeStruct((L, OUT), jnp.bfloat16),
            jax.ShapeDtypeStruct((L, OUT), jnp.bfloat16),
        ],
    )(x, wqt, wkt, wvt, cos2, sin2)

    out = pl.pallas_call(
        _attn_body,
        grid=(NBLK,),
        in_specs=[
            pl.BlockSpec((BLK, OUT), lambda i: (i, 0)),
            pl.BlockSpec((L, OUT), lambda i: (0, 0)),
            pl.BlockSpec((L, OUT), lambda i: (0, 0)),
            pl.BlockSpec((OUT, OUT), lambda i: (0, 0)),
            pl.BlockSpec((BLK, 1), lambda i: (i, 0)),
        ],
        out_specs=pl.BlockSpec((BLK, OUT), lambda i: (i, 0)),
        out_shape=jax.ShapeDtypeStruct((L, OUT), jnp.float32),
        scratch_shapes=[pltpu.VMEM((BLK, OUT), jnp.float32)],
    )(q, k, v, wot, maskf)

    return out.reshape(B, L, OUT)


# speedup vs baseline: 2.7742x; 1.2340x over previous
import functools

import jax
import jax.numpy as jnp
from jax.experimental import pallas as pl
from jax.experimental.pallas import tpu as pltpu

B, L, D_MODEL = 1, 2048, 768
H, KV, HD = 12, 12, 64
OUT = H * HD
BLK = 256
NBLK = L // BLK
SCALE = 1.0 / (HD ** 0.5)


def _rotary_cols(x, cos_c, sin_c):
    rl = jnp.concatenate([x[:, 32:], x[:, :32]], axis=1)
    rr = jnp.concatenate([x[:, -32:], x[:, :-32]], axis=1)
    lane = jax.lax.broadcasted_iota(jnp.int32, x.shape, 1)
    first_half = (lane % HD) < (HD // 2)
    roth = jnp.where(first_half, -rl, rr)
    return x * cos_c + roth * sin_c


def _tdot(a, b):
    return jax.lax.dot_general(a, b, (((1,), (1,)), ((), ())),
                               preferred_element_type=jnp.float32)


def _qkv_body(x_ref, w_ref, cosq_ref, sinq_ref, cos_ref, sin_ref,
              q_ref, k_ref, ve_ref):
    xb = x_ref[...].astype(jnp.bfloat16)
    cosq_c = jnp.concatenate([cosq_ref[...]] * H, axis=1)
    sinq_c = jnp.concatenate([sinq_ref[...]] * H, axis=1)
    cos_c = jnp.concatenate([cos_ref[...]] * H, axis=1)
    sin_c = jnp.concatenate([sin_ref[...]] * H, axis=1)
    q = _tdot(xb, w_ref[:OUT, :])
    q_ref[...] = _rotary_cols(q, cosq_c, sinq_c)
    k = _tdot(xb, w_ref[OUT:2 * OUT, :])
    k_ref[...] = _rotary_cols(k, cos_c, sin_c).astype(jnp.bfloat16)
    v = _tdot(xb, w_ref[2 * OUT:, :]).astype(jnp.bfloat16)
    ones = jnp.ones((v.shape[0], HD), jnp.bfloat16)
    pieces = []
    for h in range(H):
        pieces.append(v[:, HD * h:HD * (h + 1)])
        pieces.append(ones)
    ve_ref[...] = jnp.concatenate(pieces, axis=1)


def _attn_body(q_ref, k_ref, ve_ref, wo_ref, mask_ref, out_ref, acc_ref):
    qb = q_ref[...].astype(jnp.bfloat16)
    for h in range(H):
        s = _tdot(qb[:, HD * h:HD * (h + 1)],
                  k_ref[:, HD * h:HD * (h + 1)])
        p = jnp.exp(s.astype(jnp.bfloat16))
        o2 = jnp.dot(p, ve_ref[:, 2 * HD * h:2 * HD * (h + 1)],
                     preferred_element_type=jnp.float32)
        acc_ref[:, HD * h:HD * (h + 1)] = o2[:, :HD] / o2[:, HD:HD + 1]
    ob = acc_ref[...].astype(jnp.bfloat16)
    out_ref[...] = _tdot(ob, wo_ref[...]) * mask_ref[...]


@jax.jit
def kernel(cos, sin, hidden_states, active_mask, Wqkv, Wo):
    x = hidden_states[0]
    w_all = Wqkv.astype(jnp.bfloat16)
    wo = Wo.astype(jnp.bfloat16)
    cos2, sin2 = cos[0], sin[0]
    cosq, sinq = cos2 * SCALE, sin2 * SCALE
    maskf = active_mask[0].astype(jnp.float32).reshape(L, 1)

    q, k, ve = pl.pallas_call(
        _qkv_body,
        grid=(NBLK,),
        in_specs=[
            pl.BlockSpec((BLK, D_MODEL), lambda i: (i, 0)),
            pl.BlockSpec((3 * OUT, D_MODEL), lambda i: (0, 0)),
            pl.BlockSpec((BLK, HD), lambda i: (i, 0)),
            pl.BlockSpec((BLK, HD), lambda i: (i, 0)),
            pl.BlockSpec((BLK, HD), lambda i: (i, 0)),
            pl.BlockSpec((BLK, HD), lambda i: (i, 0)),
        ],
        out_specs=[
            pl.BlockSpec((BLK, OUT), lambda i: (i, 0)),
            pl.BlockSpec((BLK, OUT), lambda i: (i, 0)),
            pl.BlockSpec((BLK, 2 * OUT), lambda i: (i, 0)),
        ],
        out_shape=[
            jax.ShapeDtypeStruct((L, OUT), jnp.float32),
            jax.ShapeDtypeStruct((L, OUT), jnp.bfloat16),
            jax.ShapeDtypeStruct((L, 2 * OUT), jnp.bfloat16),
        ],
    )(x, w_all, cosq, sinq, cos2, sin2)

    out = pl.pallas_call(
        _attn_body,
        grid=(NBLK,),
        in_specs=[
            pl.BlockSpec((BLK, OUT), lambda i: (i, 0)),
            pl.BlockSpec((L, OUT), lambda i: (0, 0)),
            pl.BlockSpec((L, 2 * OUT), lambda i: (0, 0)),
            pl.BlockSpec((OUT, OUT), lambda i: (0, 0)),
            pl.BlockSpec((BLK, 1), lambda i: (i, 0)),
        ],
        out_specs=pl.BlockSpec((BLK, OUT), lambda i: (i, 0)),
        out_shape=jax.ShapeDtypeStruct((L, OUT), jnp.float32),
        scratch_shapes=[pltpu.VMEM((BLK, OUT), jnp.float32)],
    )(q, k, ve, wo, maskf)

    return out.reshape(B, L, OUT)
